# baseline (speedup 1.0000x reference)
import functools

import jax
import jax.numpy as jnp
from jax.experimental import pallas as pl
from jax.experimental.pallas import tpu as pltpu

N = 8192
BLK = 512
CHAIN_BLK = 1024
ADJHAT_BLK = 512
_VMEM_LIMIT = 63 * 1024 * 1024
E = 256


def _layer1_body(feat_ref, w_ref, adj_ref, out_ref, adjh_ref, s_ref):
    @pl.when(pl.program_id(0) == 0)
    def _():
        s_ref[...] = jnp.tanh(
            jnp.dot(feat_ref[...], w_ref[...], preferred_element_type=jnp.float32))

    a = adj_ref[...]
    adjh_ref[...] = a.astype(jnp.bfloat16)
    out_ref[...] = jnp.dot(a, s_ref[...], preferred_element_type=jnp.float32)


def _layer1(feat, W, adj):
    n, f = feat.shape
    e = W.shape[1]
    return pl.pallas_call(
        _layer1_body,
        grid=(n // BLK,),
        in_specs=[
            pl.BlockSpec((n, f), lambda i: (0, 0)),
            pl.BlockSpec((f, e), lambda i: (0, 0)),
            pl.BlockSpec((BLK, n), lambda i: (i, 0)),
        ],
        out_specs=[
            pl.BlockSpec((BLK, e), lambda i: (i, 0)),
            pl.BlockSpec((BLK, n), lambda i: (i, 0)),
        ],
        out_shape=[
            jax.ShapeDtypeStruct((n, e), jnp.float32),
            jax.ShapeDtypeStruct((n, n), jnp.bfloat16),
        ],
        scratch_shapes=[pltpu.VMEM((n, e), jnp.float32)],
    )(feat, W, adj)


def _chain_body(feat_ref, w_ref, adj_ref, zi_ref, zh_ref,
                zprev_ref, s_ref, zi_keep_ref, *, lin_layer, e_zi, e_zh):
    l = pl.program_id(0)
    i = pl.program_id(1)
    f = feat_ref.shape[1]
    blk = adj_ref.shape[0]

    @pl.when((l == 0) & (i == 0))
    def _():
        zprev_ref[:, :f] = feat_ref[...]
        if f < E:
            zprev_ref[:, f:] = jnp.zeros_like(zprev_ref[:, f:])

    @pl.when(i == 0)
    def _():
        s = jnp.dot(zprev_ref[...], w_ref[0], preferred_element_type=jnp.float32)
        s = jnp.where(l == lin_layer, s, jnp.tanh(s))
        s_ref[...] = s.astype(jnp.bfloat16)

    z = jnp.dot(adj_ref[...], s_ref[...], preferred_element_type=jnp.float32)
    zprev_ref[pl.ds(i * blk, blk), :] = z

    @pl.when(l == 1)
    def _():
        zi_keep_ref[pl.ds(i * blk, blk), :] = z[:, :e_zi]

    zi_ref[...] = zi_keep_ref[pl.ds(i * blk, blk), :]
    zh_ref[...] = z[:, :e_zh]


def _layer_chain(feat, w_stack, adjh, lin_layer, e_zi, e_zh):
    n, f = feat.shape
    nl = w_stack.shape[0]
    return pl.pallas_call(
        functools.partial(_chain_body, lin_layer=lin_layer, e_zi=e_zi, e_zh=e_zh),
        grid=(nl, n // CHAIN_BLK),
        in_specs=[
            pl.BlockSpec((n, f), lambda l, i: (0, 0)),
            pl.BlockSpec((1, E, E), lambda l, i: (l, 0, 0)),
            pl.BlockSpec((CHAIN_BLK, n), lambda l, i: (i, 0)),
        ],
        out_specs=[
            pl.BlockSpec((CHAIN_BLK, e_zi), lambda l, i: (i, 0)),
            pl.BlockSpec((CHAIN_BLK, e_zh), lambda l, i: (i, 0)),
        ],
        out_shape=[
            jax.ShapeDtypeStruct((n, e_zi), jnp.float32),
            jax.ShapeDtypeStruct((n, e_zh), jnp.float32),
        ],
        scratch_shapes=[
            pltpu.VMEM((n, E), jnp.float32),
            pltpu.VMEM((n, E), jnp.bfloat16),
            pltpu.VMEM((n, e_zi), jnp.float32),
        ],
        compiler_params=pltpu.CompilerParams(vmem_limit_bytes=_VMEM_LIMIT),
    )(feat, w_stack, adjh)


def _adjhat_body(zi_blk_ref, zh_blk_ref, zi_ref, zh_ref, out_ref):
    dn = (((1,), (1,)), ((), ()))
    a = jax.lax.dot_general(zi_blk_ref[...] * 0.5, zi_ref[...], dn,
                            preferred_element_type=jnp.float32)
    b = jax.lax.dot_general(zh_blk_ref[...] * 0.5, zh_ref[...], dn,
                            preferred_element_type=jnp.float32)
    out_ref[...] = (jnp.tanh(a) + jnp.tanh(b)) * 0.5 + 1.0


def _adj_hat(z_igae, z_hat):
    n, e1 = z_igae.shape
    e2 = z_hat.shape[1]
    return pl.pallas_call(
        _adjhat_body,
        grid=(n // ADJHAT_BLK,),
        in_specs=[
            pl.BlockSpec((ADJHAT_BLK, e1), lambda i: (i, 0)),
            pl.BlockSpec((ADJHAT_BLK, e2), lambda i: (i, 0)),
            pl.BlockSpec((n, e1), lambda i: (0, 0)),
            pl.BlockSpec((n, e2), lambda i: (0, 0)),
        ],
        out_specs=pl.BlockSpec((ADJHAT_BLK, n), lambda i: (i, 0)),
        out_shape=jax.ShapeDtypeStruct((n, n), jnp.float32),
        compiler_params=pltpu.CompilerParams(vmem_limit_bytes=_VMEM_LIMIT),
    )(z_igae, z_hat, z_igae, z_hat)


def _pad_w(W):
    f, e = W.shape
    return jnp.pad(W, ((0, E - f), (0, E - e)))


def kernel(x, adj, W1, W2, W3, W4, W5, W6):
    z1, adjh = _layer1(x, W1, adj)
    w_stack = jnp.stack([_pad_w(W2), _pad_w(W3), _pad_w(W4), _pad_w(W5),
                         _pad_w(W6)])
    z_igae, z_hat = _layer_chain(z1, w_stack, adjh, lin_layer=1,
                                 e_zi=W3.shape[1], e_zh=W6.shape[1])
    adj_hat = _adj_hat(z_igae, z_hat)
    return (z_igae, z_hat, adj_hat)


# speedup vs baseline: 1.0054x; 1.0054x over previous
import functools

import jax
import jax.numpy as jnp
from jax.experimental import pallas as pl
from jax.experimental.pallas import tpu as pltpu

N = 8192
BLK = 512
CHAIN_BLK = 1024
ADJHAT_BLK = 512
E = 256


def _layer1_body(feat_ref, w_ref, adj_ref, out_ref, adjh_ref, s_ref):
    @pl.when(pl.program_id(0) == 0)
    def _():
        s_ref[...] = jnp.tanh(
            jnp.dot(feat_ref[...], w_ref[...], preferred_element_type=jnp.float32))

    a = adj_ref[...]
    adjh_ref[...] = a.astype(jnp.bfloat16)
    out_ref[...] = jnp.dot(a, s_ref[...], preferred_element_type=jnp.float32)


def _layer1(feat, W, adj):
    n, f = feat.shape
    e = W.shape[1]
    return pl.pallas_call(
        _layer1_body,
        grid=(n // BLK,),
        in_specs=[
            pl.BlockSpec((n, f), lambda i: (0, 0)),
            pl.BlockSpec((f, e), lambda i: (0, 0)),
            pl.BlockSpec((BLK, n), lambda i: (i, 0)),
        ],
        out_specs=[
            pl.BlockSpec((BLK, e), lambda i: (i, 0)),
            pl.BlockSpec((BLK, n), lambda i: (i, 0)),
        ],
        out_shape=[
            jax.ShapeDtypeStruct((n, e), jnp.float32),
            jax.ShapeDtypeStruct((n, n), jnp.bfloat16),
        ],
        scratch_shapes=[pltpu.VMEM((n, e), jnp.float32)],
    )(feat, W, adj)


def _chain_body(feat_ref, w0_ref, wn_ref, adj_ref, zi_ref, zh_ref,
                s_ref, zi_keep_ref, *, lin_layer, e_zi, e_zh):
    l = pl.program_id(0)
    i = pl.program_id(1)
    blk = adj_ref.shape[0]

    @pl.when((l == 0) & (i == 0))
    def _():
        s0 = jnp.dot(feat_ref[...], w0_ref[...],
                     preferred_element_type=jnp.float32)
        s_ref[0] = jnp.tanh(s0).astype(jnp.bfloat16)

    z = jnp.dot(adj_ref[...], s_ref[l % 2], preferred_element_type=jnp.float32)

    s_nxt = jnp.dot(z, wn_ref[0], preferred_element_type=jnp.float32)
    s_nxt = jnp.where(l + 1 == lin_layer, s_nxt, jnp.tanh(s_nxt))
    s_ref[(l + 1) % 2, pl.ds(i * blk, blk), :] = s_nxt.astype(jnp.bfloat16)

    @pl.when(l == 1)
    def _():
        zi_keep_ref[pl.ds(i * blk, blk), :] = z[:, :e_zi]

    zi_ref[...] = zi_keep_ref[pl.ds(i * blk, blk), :]
    zh_ref[...] = z[:, :e_zh]


def _layer_chain(feat, w0, wn_stack, adjh, lin_layer, e_zi, e_zh):
    n, f = feat.shape
    nl = wn_stack.shape[0] + 1
    nw = wn_stack.shape[0]
    return pl.pallas_call(
        functools.partial(_chain_body, lin_layer=lin_layer, e_zi=e_zi, e_zh=e_zh),
        grid=(nl, n // CHAIN_BLK),
        in_specs=[
            pl.BlockSpec((n, f), lambda l, i: (0, 0)),
            pl.BlockSpec((f, E), lambda l, i: (0, 0)),
            pl.BlockSpec((1, E, E), lambda l, i: (jnp.minimum(l, nw - 1), 0, 0)),
            pl.BlockSpec((CHAIN_BLK, n), lambda l, i: (i, 0)),
        ],
        out_specs=[
            pl.BlockSpec((CHAIN_BLK, e_zi), lambda l, i: (i, 0)),
            pl.BlockSpec((CHAIN_BLK, e_zh), lambda l, i: (i, 0)),
        ],
        out_shape=[
            jax.ShapeDtypeStruct((n, e_zi), jnp.float32),
            jax.ShapeDtypeStruct((n, e_zh), jnp.float32),
        ],
        scratch_shapes=[
            pltpu.VMEM((2, n, E), jnp.bfloat16),
            pltpu.VMEM((n, e_zi), jnp.float32),
        ],
    )(feat, w0, wn_stack, adjh)


def _adjhat_body(zi_blk_ref, zh_blk_ref, zi_ref, zh_ref, out_ref):
    dn = (((1,), (1,)), ((), ()))
    a = jax.lax.dot_general(zi_blk_ref[...] * 0.5, zi_ref[...], dn,
                            preferred_element_type=jnp.float32)
    b = jax.lax.dot_general(zh_blk_ref[...] * 0.5, zh_ref[...], dn,
                            preferred_element_type=jnp.float32)
    out_ref[...] = (jnp.tanh(a) + jnp.tanh(b)) * 0.5 + 1.0


def _adj_hat(z_igae, z_hat):
    n, e1 = z_igae.shape
    e2 = z_hat.shape[1]
    return pl.pallas_call(
        _adjhat_body,
        grid=(n // ADJHAT_BLK,),
        in_specs=[
            pl.BlockSpec((ADJHAT_BLK, e1), lambda i: (i, 0)),
            pl.BlockSpec((ADJHAT_BLK, e2), lambda i: (i, 0)),
            pl.BlockSpec((n, e1), lambda i: (0, 0)),
            pl.BlockSpec((n, e2), lambda i: (0, 0)),
        ],
        out_specs=pl.BlockSpec((ADJHAT_BLK, n), lambda i: (i, 0)),
        out_shape=jax.ShapeDtypeStruct((n, n), jnp.float32),
    )(z_igae, z_hat, z_igae, z_hat)


def _pad_w(W):
    f, e = W.shape
    return jnp.pad(W, ((0, E - f), (0, E - e)))


def kernel(x, adj, W1, W2, W3, W4, W5, W6):
    z1, adjh = _layer1(x, W1, adj)
    w0 = jnp.pad(W2, ((0, 0), (0, E - W2.shape[1])))
    wn_stack = jnp.stack([_pad_w(W3), _pad_w(W4), _pad_w(W5), _pad_w(W6)])
    z_igae, z_hat = _layer_chain(z1, w0, wn_stack, adjh, lin_layer=1,
                                 e_zi=W3.shape[1], e_zh=W6.shape[1])
    adj_hat = _adj_hat(z_igae, z_hat)
    return (z_igae, z_hat, adj_hat)
